# baseline (speedup 1.0000x reference)
import functools

import jax
import jax.numpy as jnp
from jax import lax
from jax.experimental import pallas as pl
from jax.experimental.pallas import tpu as pltpu
from jax.experimental.pallas import tpu_sc as plsc

N = 10000
E = 320000
ROW_BLK = 1000

NW = 32
NPT = 320
NPAD = NW * NPT
CAP = 10880
CE = 2560


def _dense_a_body(h_ref, wp_t_ref, bp_ref, ws_t_ref, m_ref, hs_ref):
    h = h_ref[...]
    m_ref[...] = jnp.maximum(
        jnp.dot(h, wp_t_ref[...], preferred_element_type=jnp.float32)
        + bp_ref[...][None, :], 0.0).astype(m_ref.dtype)
    hs_ref[...] = jnp.dot(h, ws_t_ref[...], preferred_element_type=jnp.float32)


def _dense_a(h, Wp, bp, Ws, m_dtype):
    d_in = h.shape[1]
    d_m = Wp.shape[0]
    d_s = Ws.shape[0]
    return pl.pallas_call(
        _dense_a_body,
        grid=(N // ROW_BLK,),
        in_specs=[
            pl.BlockSpec((ROW_BLK, d_in), lambda i: (i, 0)),
            pl.BlockSpec((d_in, d_m), lambda i: (0, 0)),
            pl.BlockSpec((d_m,), lambda i: (0,)),
            pl.BlockSpec((d_in, d_s), lambda i: (0, 0)),
        ],
        out_specs=[
            pl.BlockSpec((ROW_BLK, d_m), lambda i: (i, 0)),
            pl.BlockSpec((ROW_BLK, d_s), lambda i: (i, 0)),
        ],
        out_shape=[
            jax.ShapeDtypeStruct((N, d_m), m_dtype),
            jax.ShapeDtypeStruct((N, d_s), jnp.float32),
        ],
    )(h, Wp.T, bp, Ws.T)


def _dense_b_body(hs_ref, agg_ref, wn_t_ref, b_ref, out_ref):
    out_ref[...] = (
        hs_ref[...]
        + jnp.dot(agg_ref[...].astype(jnp.float32), wn_t_ref[...],
                  preferred_element_type=jnp.float32)
        + b_ref[...][None, :])


def _dense_b(hs, agg, Wn, b):
    d_a = agg.shape[1]
    d_o = Wn.shape[0]
    return pl.pallas_call(
        _dense_b_body,
        grid=(N // ROW_BLK,),
        in_specs=[
            pl.BlockSpec((ROW_BLK, d_o), lambda i: (i, 0)),
            pl.BlockSpec((ROW_BLK, d_a), lambda i: (i, 0)),
            pl.BlockSpec((d_a, d_o), lambda i: (0, 0)),
            pl.BlockSpec((d_o,), lambda i: (0,)),
        ],
        out_specs=pl.BlockSpec((ROW_BLK, d_o), lambda i: (i, 0)),
        out_shape=jax.ShapeDtypeStruct((N, d_o), jnp.float32),
    )(hs, agg, Wn.T, b)


def _dense_c_body(h2_ref, wc_t_ref, bc_ref, out_ref):
    out_ref[...] = (
        jnp.dot(h2_ref[...], wc_t_ref[...], preferred_element_type=jnp.float32)
        + bc_ref[...][None, :])


def _dense_c(h2, Wc, bc):
    d_in = h2.shape[1]
    d_o = Wc.shape[0]
    return pl.pallas_call(
        _dense_c_body,
        grid=(N // ROW_BLK,),
        in_specs=[
            pl.BlockSpec((ROW_BLK, d_in), lambda i: (i, 0)),
            pl.BlockSpec((d_in, d_o), lambda i: (0, 0)),
            pl.BlockSpec((d_o,), lambda i: (0,)),
        ],
        out_specs=pl.BlockSpec((ROW_BLK, d_o), lambda i: (i, 0)),
        out_shape=jax.ShapeDtypeStruct((N, d_o), jnp.float32),
    )(h2, Wc.T, bc)


def _dense_mid_body(hs_ref, agg_ref, wn_t_ref, b_ref, wp_t_ref, bp_ref,
                    ws_t_ref, h1_ref, m_ref, hs2_ref):
    h1 = (hs_ref[...]
          + jnp.dot(agg_ref[...].astype(jnp.float32), wn_t_ref[...],
                    preferred_element_type=jnp.float32)
          + b_ref[...][None, :])
    h1_ref[...] = h1
    m_ref[...] = jnp.maximum(
        jnp.dot(h1, wp_t_ref[...], preferred_element_type=jnp.float32)
        + bp_ref[...][None, :], 0.0).astype(m_ref.dtype)
    hs2_ref[...] = jnp.dot(h1, ws_t_ref[...],
                           preferred_element_type=jnp.float32)


def _dense_mid(hs, agg, Wn, b, Wp, bp, Ws, m_dtype):
    d_a = agg.shape[1]
    d_o = Wn.shape[0]
    d_m = Wp.shape[0]
    d_s = Ws.shape[0]
    return pl.pallas_call(
        _dense_mid_body,
        grid=(N // ROW_BLK,),
        in_specs=[
            pl.BlockSpec((ROW_BLK, d_o), lambda i: (i, 0)),
            pl.BlockSpec((ROW_BLK, d_a), lambda i: (i, 0)),
            pl.BlockSpec((d_a, d_o), lambda i: (0, 0)),
            pl.BlockSpec((d_o,), lambda i: (0,)),
            pl.BlockSpec((d_o, d_m), lambda i: (0, 0)),
            pl.BlockSpec((d_m,), lambda i: (0,)),
            pl.BlockSpec((d_o, d_s), lambda i: (0, 0)),
        ],
        out_specs=[
            pl.BlockSpec((ROW_BLK, d_o), lambda i: (i, 0)),
            pl.BlockSpec((ROW_BLK, d_m), lambda i: (i, 0)),
            pl.BlockSpec((ROW_BLK, d_s), lambda i: (i, 0)),
        ],
        out_shape=[
            jax.ShapeDtypeStruct((N, d_o), jnp.float32),
            jax.ShapeDtypeStruct((N, d_m), m_dtype),
            jax.ShapeDtypeStruct((N, d_s), jnp.float32),
        ],
    )(hs, agg, Wn.T, b, Wp.T, bp, Ws.T)


def _dense_tail_body(hs_ref, agg_ref, wn_t_ref, b_ref, wc_t_ref, bc_ref,
                     h2_ref, lg_ref):
    h2 = (hs_ref[...]
          + jnp.dot(agg_ref[...].astype(jnp.float32), wn_t_ref[...],
                    preferred_element_type=jnp.float32)
          + b_ref[...][None, :])
    h2_ref[...] = h2
    lg_ref[...] = (jnp.dot(h2, wc_t_ref[...],
                           preferred_element_type=jnp.float32)
                   + bc_ref[...][None, :])


def _dense_tail(hs, agg, Wn, b, Wc, bc):
    d_a = agg.shape[1]
    d_o = Wn.shape[0]
    d_c = Wc.shape[0]
    return pl.pallas_call(
        _dense_tail_body,
        grid=(N // ROW_BLK,),
        in_specs=[
            pl.BlockSpec((ROW_BLK, d_o), lambda i: (i, 0)),
            pl.BlockSpec((ROW_BLK, d_a), lambda i: (i, 0)),
            pl.BlockSpec((d_a, d_o), lambda i: (0, 0)),
            pl.BlockSpec((d_o,), lambda i: (0,)),
            pl.BlockSpec((d_o, d_c), lambda i: (0, 0)),
            pl.BlockSpec((d_c,), lambda i: (0,)),
        ],
        out_specs=[
            pl.BlockSpec((ROW_BLK, d_o), lambda i: (i, 0)),
            pl.BlockSpec((ROW_BLK, d_c), lambda i: (i, 0)),
        ],
        out_shape=[
            jax.ShapeDtypeStruct((N, d_o), jnp.float32),
            jax.ShapeDtypeStruct((N, d_c), jnp.float32),
        ],
    )(hs, agg, Wn.T, b, Wc.T, bc)


def _make_segmax_sc(D, G, packed):
    if packed:
        nvec, lanes, adt = D // 32, 32, jnp.bfloat16
        tbl_t = jax.ShapeDtypeStruct((N, D // 2), jnp.int32)
        row_t = pltpu.VMEM((2, G, D // 2), jnp.int32)
    else:
        nvec, lanes, adt = D // 16, 16, jnp.float32
        tbl_t = jax.ShapeDtypeStruct((N, D), jnp.float32)
        row_t = pltpu.VMEM((2, G, D), jnp.float32)
    del tbl_t
    mesh = plsc.VectorSubcoreMesh(core_axis_name="c", subcore_axis_name="s")

    @functools.partial(
        pl.kernel, mesh=mesh,
        compiler_params=pltpu.CompilerParams(needs_layout_passes=False),
        out_type=jax.ShapeDtypeStruct((NPAD, D), adt),
        scratch_types=[
            pltpu.VMEM((CAP + 16,), jnp.int32),
            pltpu.VMEM((CAP + 16,), jnp.int32),
            pltpu.VMEM((CE,), jnp.int32),
            pltpu.VMEM((CE,), jnp.int32),
            pltpu.VMEM((NPT + 1, D), adt),
            row_t,
            pltpu.SemaphoreType.DMA,
            pltpu.SemaphoreType.DMA,
        ],
    )
    def seg(m_hbm, src_hbm, dst_hbm, out_hbm,
            srcl, dstl, sbuf, dbuf, agg, rows, sem0, sem1):
        wid = lax.axis_index("s") * 2 + lax.axis_index("c")
        lo = wid * NPT

        def init_agg(r, _):
            for j in range(nvec):
                agg[r, pl.ds(j * lanes, lanes)] = jnp.zeros((lanes,), adt)
            return 0
        lax.fori_loop(0, NPT + 1, init_agg, 0)

        def init_lists(i, _):
            srcl[pl.ds(i * 16, 16)] = jnp.zeros((16,), jnp.int32)
            dstl[pl.ds(i * 16, 16)] = jnp.full((16,), NPT, jnp.int32)
            return 0
        lax.fori_loop(0, (CAP + 16) // 16, init_lists, 0)

        def chunk_body(c, off_vec):
            pltpu.sync_copy(src_hbm.at[pl.ds(c * CE, CE)], sbuf)
            pltpu.sync_copy(dst_hbm.at[pl.ds(c * CE, CE)], dbuf)

            @plsc.parallel_loop(0, CE // 16, 1, unroll=16, carry=off_vec)
            def group_body(g, off):
                d16 = dbuf[pl.ds(g * 16, 16)]
                s16 = sbuf[pl.ds(g * 16, 16)]
                dloc = d16 - lo
                mask = dloc.astype(jnp.uint32) < jnp.uint32(NPT)
                ones = jnp.where(mask, jnp.int32(1), jnp.int32(0))
                idx = jnp.minimum(off + plsc.cumsum(ones) - 1, CAP - 1)
                plsc.store_scatter(srcl, [idx], s16, mask=mask)
                plsc.store_scatter(dstl, [idx], dloc, mask=mask)
                return off + plsc.all_reduce_population_count(mask)

            return group_body

        off_vec = lax.fori_loop(0, E // CE, chunk_body,
                                jnp.zeros((16,), jnp.int32))
        n = jnp.minimum(off_vec[0], CAP)
        nch = (n + (G - 1)) // G

        def start(c, slot, sem):
            pltpu.async_copy(m_hbm.at[srcl.at[pl.ds(c * G, G)]],
                             rows.at[slot], sem)

        def wait(c, slot, sem):
            pltpu.make_async_copy(m_hbm.at[srcl.at[pl.ds(c * G, G)]],
                                  rows.at[slot], sem).wait()

        def process(c, slot):
            def edge_body(e, _):
                dl = dstl[pl.ds(c * G + e, 16)][0]
                for j in range(nvec):
                    sl = pl.ds(j * lanes, lanes)
                    if packed:
                        x = plsc.bitcast(rows[slot, e, pl.ds(j * 16, 16)],
                                         jnp.bfloat16)
                    else:
                        x = rows[slot, e, sl]
                    agg[dl, sl] = jnp.maximum(agg[dl, sl], x)
                return 0
            lax.fori_loop(0, G, edge_body, 0, unroll=8)

        @pl.when(nch > 0)
        def _():
            start(0, 0, sem0)

        @pl.when(nch > 1)
        def _():
            start(1, 1, sem1)

        def pair_body(c2, _):
            a = 2 * c2
            wait(a, 0, sem0)
            process(a, 0)

            @pl.when(a + 2 < nch)
            def _():
                start(a + 2, 0, sem0)

            @pl.when(a + 1 < nch)
            def _():
                wait(a + 1, 1, sem1)
                process(a + 1, 1)

                @pl.when(a + 3 < nch)
                def _():
                    start(a + 3, 1, sem1)
            return 0
        lax.fori_loop(0, (nch + 1) // 2, pair_body, 0)

        pltpu.sync_copy(agg.at[pl.ds(0, NPT)], out_hbm.at[pl.ds(lo, NPT)])

    return seg


_segmax_128 = _make_segmax_sc(128, 128, packed=False)
_segmax_256 = _make_segmax_sc(256, 32, packed=False)


def _segmax(m, edge_index):
    seg = _segmax_128 if m.shape[1] == 128 else _segmax_256
    agg = seg(m, edge_index[0], edge_index[1])
    return agg[:N]


def kernel(h, edge_index_block0, edge_index_block1,
           Wp1, bp1, Ws1, Wn1, b1,
           Wp2, bp2, Ws2, Wn2, b2,
           Wc, bc):
    m1, hs1 = _dense_a(h, Wp1, bp1, Ws1, jnp.float32)
    agg1 = _segmax(m1, edge_index_block0)
    h1, m2, hs2 = _dense_mid(hs1, agg1, Wn1, b1, Wp2, bp2, Ws2, jnp.float32)
    agg2 = _segmax(m2, edge_index_block1)
    h2, logits = _dense_tail(hs2, agg2, Wn2, b2, Wc, bc)
    return (h2, logits)


# speedup vs baseline: 1.0381x; 1.0381x over previous
import functools

import jax
import jax.numpy as jnp
from jax import lax
from jax.experimental import pallas as pl
from jax.experimental.pallas import tpu as pltpu
from jax.experimental.pallas import tpu_sc as plsc

N = 10000
E = 320000
ROW_BLK = 1000

NW = 32
NPT = 320
NPAD = NW * NPT
CAP = 10880
CE = 2560


def _dense_a_body(h_ref, wp_t_ref, bp_ref, ws_t_ref, m_ref, hs_ref):
    h = h_ref[...]
    m_ref[...] = jnp.maximum(
        jnp.dot(h, wp_t_ref[...], preferred_element_type=jnp.float32)
        + bp_ref[...][None, :], 0.0).astype(m_ref.dtype)
    hs_ref[...] = jnp.dot(h, ws_t_ref[...], preferred_element_type=jnp.float32)


def _dense_a(h, Wp, bp, Ws, m_dtype):
    d_in = h.shape[1]
    d_m = Wp.shape[0]
    d_s = Ws.shape[0]
    return pl.pallas_call(
        _dense_a_body,
        grid=(N // ROW_BLK,),
        in_specs=[
            pl.BlockSpec((ROW_BLK, d_in), lambda i: (i, 0)),
            pl.BlockSpec((d_in, d_m), lambda i: (0, 0)),
            pl.BlockSpec((d_m,), lambda i: (0,)),
            pl.BlockSpec((d_in, d_s), lambda i: (0, 0)),
        ],
        out_specs=[
            pl.BlockSpec((ROW_BLK, d_m), lambda i: (i, 0)),
            pl.BlockSpec((ROW_BLK, d_s), lambda i: (i, 0)),
        ],
        out_shape=[
            jax.ShapeDtypeStruct((N, d_m), m_dtype),
            jax.ShapeDtypeStruct((N, d_s), jnp.float32),
        ],
    )(h, Wp.T, bp, Ws.T)


def _dense_b_body(hs_ref, agg_ref, wn_t_ref, b_ref, out_ref):
    out_ref[...] = (
        hs_ref[...]
        + jnp.dot(agg_ref[...].astype(jnp.float32), wn_t_ref[...],
                  preferred_element_type=jnp.float32)
        + b_ref[...][None, :])


def _dense_b(hs, agg, Wn, b):
    d_a = agg.shape[1]
    d_o = Wn.shape[0]
    return pl.pallas_call(
        _dense_b_body,
        grid=(N // ROW_BLK,),
        in_specs=[
            pl.BlockSpec((ROW_BLK, d_o), lambda i: (i, 0)),
            pl.BlockSpec((ROW_BLK, d_a), lambda i: (i, 0)),
            pl.BlockSpec((d_a, d_o), lambda i: (0, 0)),
            pl.BlockSpec((d_o,), lambda i: (0,)),
        ],
        out_specs=pl.BlockSpec((ROW_BLK, d_o), lambda i: (i, 0)),
        out_shape=jax.ShapeDtypeStruct((N, d_o), jnp.float32),
    )(hs, agg, Wn.T, b)


def _dense_c_body(h2_ref, wc_t_ref, bc_ref, out_ref):
    out_ref[...] = (
        jnp.dot(h2_ref[...], wc_t_ref[...], preferred_element_type=jnp.float32)
        + bc_ref[...][None, :])


def _dense_c(h2, Wc, bc):
    d_in = h2.shape[1]
    d_o = Wc.shape[0]
    return pl.pallas_call(
        _dense_c_body,
        grid=(N // ROW_BLK,),
        in_specs=[
            pl.BlockSpec((ROW_BLK, d_in), lambda i: (i, 0)),
            pl.BlockSpec((d_in, d_o), lambda i: (0, 0)),
            pl.BlockSpec((d_o,), lambda i: (0,)),
        ],
        out_specs=pl.BlockSpec((ROW_BLK, d_o), lambda i: (i, 0)),
        out_shape=jax.ShapeDtypeStruct((N, d_o), jnp.float32),
    )(h2, Wc.T, bc)


def _dense_mid_body(hs_ref, agg_ref, wn_t_ref, b_ref, wp_t_ref, bp_ref,
                    ws_t_ref, h1_ref, m_ref, hs2_ref):
    h1 = (hs_ref[...]
          + jnp.dot(agg_ref[...].astype(jnp.float32), wn_t_ref[...],
                    preferred_element_type=jnp.float32)
          + b_ref[...][None, :])
    h1_ref[...] = h1
    m_ref[...] = jnp.maximum(
        jnp.dot(h1, wp_t_ref[...], preferred_element_type=jnp.float32)
        + bp_ref[...][None, :], 0.0).astype(m_ref.dtype)
    hs2_ref[...] = jnp.dot(h1, ws_t_ref[...],
                           preferred_element_type=jnp.float32)


def _dense_mid(hs, agg, Wn, b, Wp, bp, Ws, m_dtype):
    d_a = agg.shape[1]
    d_o = Wn.shape[0]
    d_m = Wp.shape[0]
    d_s = Ws.shape[0]
    return pl.pallas_call(
        _dense_mid_body,
        grid=(N // ROW_BLK,),
        in_specs=[
            pl.BlockSpec((ROW_BLK, d_o), lambda i: (i, 0)),
            pl.BlockSpec((ROW_BLK, d_a), lambda i: (i, 0)),
            pl.BlockSpec((d_a, d_o), lambda i: (0, 0)),
            pl.BlockSpec((d_o,), lambda i: (0,)),
            pl.BlockSpec((d_o, d_m), lambda i: (0, 0)),
            pl.BlockSpec((d_m,), lambda i: (0,)),
            pl.BlockSpec((d_o, d_s), lambda i: (0, 0)),
        ],
        out_specs=[
            pl.BlockSpec((ROW_BLK, d_o), lambda i: (i, 0)),
            pl.BlockSpec((ROW_BLK, d_m), lambda i: (i, 0)),
            pl.BlockSpec((ROW_BLK, d_s), lambda i: (i, 0)),
        ],
        out_shape=[
            jax.ShapeDtypeStruct((N, d_o), jnp.float32),
            jax.ShapeDtypeStruct((N, d_m), m_dtype),
            jax.ShapeDtypeStruct((N, d_s), jnp.float32),
        ],
    )(hs, agg, Wn.T, b, Wp.T, bp, Ws.T)


def _dense_tail_body(hs_ref, agg_ref, wn_t_ref, b_ref, wc_t_ref, bc_ref,
                     h2_ref, lg_ref):
    h2 = (hs_ref[...]
          + jnp.dot(agg_ref[...].astype(jnp.float32), wn_t_ref[...],
                    preferred_element_type=jnp.float32)
          + b_ref[...][None, :])
    h2_ref[...] = h2
    lg_ref[...] = (jnp.dot(h2, wc_t_ref[...],
                           preferred_element_type=jnp.float32)
                   + bc_ref[...][None, :])


def _dense_tail(hs, agg, Wn, b, Wc, bc):
    d_a = agg.shape[1]
    d_o = Wn.shape[0]
    d_c = Wc.shape[0]
    return pl.pallas_call(
        _dense_tail_body,
        grid=(N // ROW_BLK,),
        in_specs=[
            pl.BlockSpec((ROW_BLK, d_o), lambda i: (i, 0)),
            pl.BlockSpec((ROW_BLK, d_a), lambda i: (i, 0)),
            pl.BlockSpec((d_a, d_o), lambda i: (0, 0)),
            pl.BlockSpec((d_o,), lambda i: (0,)),
            pl.BlockSpec((d_o, d_c), lambda i: (0, 0)),
            pl.BlockSpec((d_c,), lambda i: (0,)),
        ],
        out_specs=[
            pl.BlockSpec((ROW_BLK, d_o), lambda i: (i, 0)),
            pl.BlockSpec((ROW_BLK, d_c), lambda i: (i, 0)),
        ],
        out_shape=[
            jax.ShapeDtypeStruct((N, d_o), jnp.float32),
            jax.ShapeDtypeStruct((N, d_c), jnp.float32),
        ],
    )(hs, agg, Wn.T, b, Wc.T, bc)


def _make_segmax_sc(D, G, packed):
    if packed:
        nvec, lanes, adt = D // 32, 32, jnp.bfloat16
        tbl_t = jax.ShapeDtypeStruct((N, D // 2), jnp.int32)
        row_t = pltpu.VMEM((2, G, D // 2), jnp.int32)
    else:
        nvec, lanes, adt = D // 16, 16, jnp.float32
        tbl_t = jax.ShapeDtypeStruct((N, D), jnp.float32)
        row_t = pltpu.VMEM((2, G, D), jnp.float32)
    del tbl_t
    mesh = plsc.VectorSubcoreMesh(core_axis_name="c", subcore_axis_name="s")

    @functools.partial(
        pl.kernel, mesh=mesh,
        compiler_params=pltpu.CompilerParams(needs_layout_passes=False),
        out_type=jax.ShapeDtypeStruct((NPAD, D), adt),
        scratch_types=[
            pltpu.VMEM((CAP + 16,), jnp.int32),
            pltpu.VMEM((CAP + 16,), jnp.int32),
            pltpu.VMEM((CE,), jnp.int32),
            pltpu.VMEM((CE,), jnp.int32),
            pltpu.VMEM((NPT + 1, D), adt),
            row_t,
            pltpu.SemaphoreType.DMA,
            pltpu.SemaphoreType.DMA,
        ],
    )
    def seg(m_hbm, src_hbm, dst_hbm, out_hbm,
            srcl, dstl, sbuf, dbuf, agg, rows, sem0, sem1):
        wid = lax.axis_index("s") * 2 + lax.axis_index("c")
        lo = wid * NPT

        def init_agg(r, _):
            for j in range(nvec):
                agg[r, pl.ds(j * lanes, lanes)] = jnp.zeros((lanes,), adt)
            return 0
        lax.fori_loop(0, NPT + 1, init_agg, 0)

        def init_lists(i, _):
            srcl[pl.ds(i * 16, 16)] = jnp.zeros((16,), jnp.int32)
            dstl[pl.ds(i * 16, 16)] = jnp.full((16,), NPT, jnp.int32)
            return 0
        lax.fori_loop(0, (CAP + 16) // 16, init_lists, 0)

        def chunk_body(c, off_vec):
            pltpu.sync_copy(src_hbm.at[pl.ds(c * CE, CE)], sbuf)
            pltpu.sync_copy(dst_hbm.at[pl.ds(c * CE, CE)], dbuf)

            @plsc.parallel_loop(0, CE // 16, 1, unroll=8, carry=off_vec)
            def group_body(g, off):
                d16 = dbuf[pl.ds(g * 16, 16)]
                s16 = sbuf[pl.ds(g * 16, 16)]
                dloc = d16 - lo
                mask = dloc.astype(jnp.uint32) < jnp.uint32(NPT)
                ones = jnp.where(mask, jnp.int32(1), jnp.int32(0))
                idx = jnp.minimum(off + plsc.cumsum(ones) - 1, CAP - 1)
                plsc.store_scatter(srcl, [idx], s16, mask=mask)
                plsc.store_scatter(dstl, [idx], dloc, mask=mask)
                return off + plsc.all_reduce_population_count(mask)

            return group_body

        off_vec = lax.fori_loop(0, E // CE, chunk_body,
                                jnp.zeros((16,), jnp.int32))
        n = jnp.minimum(off_vec[0], CAP)
        nch = (n + (G - 1)) // G

        def start(c, slot, sem):
            pltpu.async_copy(m_hbm.at[srcl.at[pl.ds(c * G, G)]],
                             rows.at[slot], sem)

        def wait(c, slot, sem):
            pltpu.make_async_copy(m_hbm.at[srcl.at[pl.ds(c * G, G)]],
                                  rows.at[slot], sem).wait()

        def process(c, slot):
            def edge_body(e, _):
                dl = dstl[pl.ds(c * G + e, 16)][0]
                for j in range(nvec):
                    sl = pl.ds(j * lanes, lanes)
                    if packed:
                        x = plsc.bitcast(rows[slot, e, pl.ds(j * 16, 16)],
                                         jnp.bfloat16)
                    else:
                        x = rows[slot, e, sl]
                    agg[dl, sl] = jnp.maximum(agg[dl, sl], x)
                return 0
            lax.fori_loop(0, G, edge_body, 0, unroll=4)

        @pl.when(nch > 0)
        def _():
            start(0, 0, sem0)

        @pl.when(nch > 1)
        def _():
            start(1, 1, sem1)

        def pair_body(c2, _):
            a = 2 * c2
            wait(a, 0, sem0)
            process(a, 0)

            @pl.when(a + 2 < nch)
            def _():
                start(a + 2, 0, sem0)

            @pl.when(a + 1 < nch)
            def _():
                wait(a + 1, 1, sem1)
                process(a + 1, 1)

                @pl.when(a + 3 < nch)
                def _():
                    start(a + 3, 1, sem1)
            return 0
        lax.fori_loop(0, (nch + 1) // 2, pair_body, 0)

        pltpu.sync_copy(agg.at[pl.ds(0, NPT)], out_hbm.at[pl.ds(lo, NPT)])

    return seg


_segmax_128 = _make_segmax_sc(128, 64, packed=False)
_segmax_256 = _make_segmax_sc(256, 32, packed=False)


def _segmax(m, edge_index):
    seg = _segmax_128 if m.shape[1] == 128 else _segmax_256
    agg = seg(m, edge_index[0], edge_index[1])
    return agg[:N]


def kernel(h, edge_index_block0, edge_index_block1,
           Wp1, bp1, Ws1, Wn1, b1,
           Wp2, bp2, Ws2, Wn2, b2,
           Wc, bc):
    m1, hs1 = _dense_a(h, Wp1, bp1, Ws1, jnp.float32)
    agg1 = _segmax(m1, edge_index_block0)
    h1, m2, hs2 = _dense_mid(hs1, agg1, Wn1, b1, Wp2, bp2, Ws2, jnp.float32)
    agg2 = _segmax(m2, edge_index_block1)
    h2, logits = _dense_tail(hs2, agg2, Wn2, b2, Wc, bc)
    return (h2, logits)
